# baseline (speedup 1.0000x reference)
import functools

import jax
import jax.numpy as jnp
from jax import lax
from jax.experimental import pallas as pl
from jax.experimental.pallas import tpu as pltpu
from jax.experimental.pallas import tpu_sc as plsc

_MU = 3.5
_L = 16
_GROUP = 128
_CHUNK = 256


@functools.lru_cache(maxsize=None)
def _build(B: int, K: int):
    rows_per_group = _GROUP // K
    info = plsc.get_sparse_core_info()
    nw = info.num_cores * info.num_subcores
    assert B % (nw * _CHUNK) == 0 or B % nw == 0
    bpw = B // nw
    n_chunks = bpw // _CHUNK
    mesh = plsc.VectorSubcoreMesh(core_axis_name="c", subcore_axis_name="s")

    @functools.partial(
        pl.kernel,
        mesh=mesh,
        out_type=jax.ShapeDtypeStruct((B,), jnp.float32),
        compiler_params=pltpu.CompilerParams(needs_layout_passes=False),
        scratch_types=[
            pltpu.VMEM((bpw,), jnp.int32),
            pltpu.VMEM((bpw,), jnp.int32),
            pltpu.VMEM((_CHUNK,), jnp.int32),
            pltpu.VMEM((_CHUNK,), jnp.int32),
            pltpu.VMEM((_CHUNK, _GROUP), jnp.float32),
            pltpu.VMEM((_CHUNK, _GROUP), jnp.float32),
            pltpu.VMEM((bpw,), jnp.float32),
            pltpu.VMEM((bpw,), jnp.float32),
            pltpu.VMEM((bpw,), jnp.float32),
            pltpu.SemaphoreType.DMA,
            pltpu.SemaphoreType.DMA,
            pltpu.SemaphoreType.DMA,
            pltpu.SemaphoreType.DMA,
        ],
    )
    def fwd(uidx_hbm, iidx_hbm, p_hbm, q_hbm, bu_hbm, bi_hbm, out_hbm,
            uidx_v, iidx_v, ug_v, ig_v, p_rows, q_rows, bu_v, bi_v, out_v,
            sem_p, sem_q, sem_bu, sem_bi):
        wid = lax.axis_index("s") * info.num_cores + lax.axis_index("c")
        base = wid * bpw

        pltpu.sync_copy(uidx_hbm.at[pl.ds(base, bpw)], uidx_v)
        pltpu.sync_copy(iidx_hbm.at[pl.ds(base, bpw)], iidx_v)

        cp_bu = pltpu.async_copy(bu_hbm.at[uidx_v], bu_v, sem_bu)
        cp_bi = pltpu.async_copy(bi_hbm.at[iidx_v], bi_v, sem_bi)

        def chunk(c, carry):
            c0 = c * _CHUNK
            for j in range(_CHUNK // _L):
                o = j * _L
                ug_v[pl.ds(o, _L)] = jnp.right_shift(
                    uidx_v[pl.ds(c0 + o, _L)], 2)
                ig_v[pl.ds(o, _L)] = jnp.right_shift(
                    iidx_v[pl.ds(c0 + o, _L)], 2)
            cp_p = pltpu.async_copy(p_hbm.at[ug_v], p_rows, sem_p)
            cp_q = pltpu.async_copy(q_hbm.at[ig_v], q_rows, sem_q)
            cp_p.wait()
            cp_q.wait()
            rows16 = lax.iota(jnp.int32, _L)
            for g in range(_CHUNK // _L):
                o = g * _L
                uu = uidx_v[pl.ds(c0 + o, _L)]
                ii = iidx_v[pl.ds(c0 + o, _L)]
                ucol0 = jnp.left_shift(jnp.bitwise_and(uu, rows_per_group - 1), 5)
                icol0 = jnp.left_shift(jnp.bitwise_and(ii, rows_per_group - 1), 5)
                rows = o + rows16
                acc = jnp.zeros((_L,), jnp.float32)
                for k in range(K):
                    pc = plsc.load_gather(p_rows, [rows, ucol0 + k])
                    qc = plsc.load_gather(q_rows, [rows, icol0 + k])
                    acc = acc + pc * qc
                out_v[pl.ds(c0 + o, _L)] = acc
            return carry

        lax.fori_loop(0, n_chunks, chunk, 0)

        cp_bu.wait()
        cp_bi.wait()
        for j in range(bpw // _L):
            o = j * _L
            out_v[pl.ds(o, _L)] = (
                _MU + bu_v[pl.ds(o, _L)] + bi_v[pl.ds(o, _L)]
                + out_v[pl.ds(o, _L)]
            )

        pltpu.sync_copy(out_v, out_hbm.at[pl.ds(base, bpw)])

    return fwd


def kernel(user_idx, item_idx, P, Q, b_u, b_i):
    B = user_idx.shape[0]
    n_users, K = P.shape
    fwd = _build(B, K)
    rows_per_group = _GROUP // K
    p_g = P.reshape(n_users // rows_per_group, _GROUP)
    q_g = Q.reshape(Q.shape[0] // rows_per_group, _GROUP)
    return fwd(user_idx.astype(jnp.int32), item_idx.astype(jnp.int32),
               p_g, q_g, b_u, b_i)
